# baseline (speedup 1.0000x reference)
import functools

import jax
import jax.numpy as jnp
from jax import lax
from jax.experimental import pallas as pl
from jax.experimental.pallas import tpu as pltpu
from jax.experimental.pallas import tpu_sc as plsc

H = 224
W = 224
NCOL = 225
N = H * NCOL
NP = 50688
C = 64
G = 65
G3 = G * G * G
GP = 276480
GCH = GP // 32
SENTR = 274688
VPAD = G3 + 7
NROWZ = N
NZROWS = 512
NF = N + NZROWS

OFFS = [(dx, dy, dz) for dx in (-1, 0, 1) for dy in (-1, 0, 1) for dz in (-1, 0, 1)]

R = 27 * NP
RCH = 256
CPK = NP // RCH
NCHUNK = R // RCH
NPAIR = (NCHUNK + 63) // 64

MB = 512
NBLK = NP // MB


def _tca_body(d_ref, v_ref, nbv_ref):
    d = d_ref[...]
    zmin = jnp.min(d)
    zmax = jnp.max(d)
    z = (d - zmin) / (zmax - zmin + 1e-08)
    jj = lax.broadcasted_iota(jnp.int32, (H, NCOL), 1)
    ii = lax.broadcasted_iota(jnp.int32, (H, NCOL), 0)
    xf = (jj - 1).astype(jnp.float32) / (W - 1)
    yf = ii.astype(jnp.float32) / (H - 1)
    xb = jnp.clip(jnp.round(xf * 64).astype(jnp.int32), 0, 64)
    yb = jnp.clip(jnp.round(yf * 64).astype(jnp.int32), 0, 64)
    zb = jnp.clip(jnp.round(z * 64).astype(jnp.int32), 0, 64)
    iscls = jj == 0
    zero = jnp.zeros((H, NCOL), jnp.int32)
    xb = jnp.where(iscls, zero, xb)
    yb = jnp.where(iscls, zero, yb)
    zb = jnp.where(iscls, zero, zb)
    v = xb * 4225 + yb * 65 + zb
    v_ref[...] = v
    sent = SENTR + ((ii * NCOL + jj) & 1023)
    for k, (dx, dy, dz) in enumerate(OFFS):
        nx, ny, nz = xb + dx, yb + dy, zb + dz
        valid = ((nx >= 0) & (nx <= 64) & (ny >= 0) & (ny <= 64)
                 & (nz >= 0) & (nz <= 64))
        nbv_ref[k] = jnp.where(valid, v + (dx * 4225 + dy * 65 + dz), sent)


_tca = pl.pallas_call(
    _tca_body,
    out_shape=(
        jax.ShapeDtypeStruct((H, NCOL), jnp.int32),
        jax.ShapeDtypeStruct((27, H, NCOL), jnp.int32),
    ),
)


def _mesh():
    return plsc.VectorSubcoreMesh(core_axis_name="c", subcore_axis_name="s")


def _wid():
    return lax.axis_index("s") * 2 + lax.axis_index("c")


def _scb_body(v_hbm, grid_hbm, v_vmem, gch, sem):
    wid = _wid()
    lo = pl.multiple_of(wid * GCH, GCH)
    pltpu.sync_copy(v_hbm, v_vmem)
    neg1 = jnp.full((16,), -1, jnp.int32)

    def ms(i, c):
        gch[pl.ds(i * 16, 16)] = neg1
        return c

    lax.fori_loop(0, GCH // 16, ms, 0)

    lane = lax.iota(jnp.int32, 16)
    lane_bit = jnp.int32(1) << lane
    ones = jnp.full((16,), 1, jnp.int32)

    def body(i, c):
        vv = v_vmem[pl.ds(i * 16, 16)]
        m = (vv >= lo) & (vv < lo + GCH)
        loc = jnp.where(m, vv - lo, 0)
        base = ones * (i * 65536)
        plsc.store_scatter(gch, [loc], base, mask=m)
        plsc.addupdate_scatter(gch, [loc], lane_bit, mask=m)
        return c

    lax.fori_loop(0, NP // 16, body, 0)

    def decode(i, c):
        s = gch[pl.ds(i * 16, 16)]
        hi = lax.shift_right_logical(s, 16)
        lb = s & 0xFFFF
        f = lb.astype(jnp.float32)
        fbits = lax.bitcast_convert_type(f, jnp.int32)
        msb = lax.shift_right_logical(fbits, 23) - 127
        gch[pl.ds(i * 16, 16)] = jnp.where(s < 0, s, hi * 16 + msb)
        return c

    lax.fori_loop(0, GCH // 16, decode, 0)
    pltpu.sync_copy(gch, grid_hbm.at[pl.ds(lo, GCH)])


def _scb(mesh):
    return pl.kernel(
        _scb_body,
        out_type=jax.ShapeDtypeStruct((GP,), jnp.int32),
        mesh=mesh,
        compiler_params=pltpu.CompilerParams(needs_layout_passes=False),
        scratch_types=[
            pltpu.VMEM((NP,), jnp.int32),
            pltpu.VMEM((GCH,), jnp.int32),
            pltpu.SemaphoreType.DMA,
        ],
    )


def _coords(cc):
    k = cc // CPK
    n0 = pl.multiple_of((cc % CPK) * RCH, RCH)
    kc = pl.multiple_of(k * 128, 128)
    r0 = pl.multiple_of(cc * RCH, RCH)
    return n0, kc, r0


def _sel(rowv):
    lane = lax.iota(jnp.int32, 16)

    def body(i, c):
        pj = rowv[pl.ds(i * 16, 16)]
        zspread = NROWZ + i * 16 + lane
        rowv[pl.ds(i * 16, 16)] = jnp.where(pj < 0, zspread, pj)
        return c

    lax.fori_loop(0, RCH // 16, body, 0)


def _scc_body(nbv_hbm, grid_hbm, feats_hbm, g_hbm,
              idxa, idxb, rowa, rowb, gva, gvb,
              spa, spb, sfa, sfb, swa, swb):
    wid = _wid()

    def pair(i, c):
        c0 = jnp.minimum(wid + 32 * (2 * i), NCHUNK - 1)
        c1 = jnp.minimum(wid + 32 * (2 * i + 1), NCHUNK - 1)
        n00, kc0, r00 = _coords(c0)
        n01, kc1, r01 = _coords(c1)
        pltpu.sync_copy(nbv_hbm.at[pl.ds(r00, RCH)], idxa)
        pa = pltpu.async_copy(grid_hbm.at[idxa], rowa, spa)
        pltpu.sync_copy(nbv_hbm.at[pl.ds(r01, RCH)], idxb)
        pb = pltpu.async_copy(grid_hbm.at[idxb], rowb, spb)
        pa.wait()
        _sel(rowa)
        fa = pltpu.async_copy(feats_hbm.at[rowa], gva, sfa)
        pb.wait()
        _sel(rowb)
        fb = pltpu.async_copy(feats_hbm.at[rowb], gvb, sfb)
        fa.wait()
        wa = pltpu.async_copy(gva, g_hbm.at[pl.ds(n00, RCH), pl.ds(kc0, 128)],
                              swa)
        fb.wait()
        wb = pltpu.async_copy(gvb, g_hbm.at[pl.ds(n01, RCH), pl.ds(kc1, 128)],
                              swb)
        wa.wait()
        wb.wait()
        return c

    lax.fori_loop(0, NPAIR, pair, 0)


def _scc(mesh):
    return pl.kernel(
        _scc_body,
        out_type=jax.ShapeDtypeStruct((NP, 27 * 128), jnp.float32),
        mesh=mesh,
        compiler_params=pltpu.CompilerParams(needs_layout_passes=False),
        scratch_types=[
            pltpu.VMEM((RCH,), jnp.int32),
            pltpu.VMEM((RCH,), jnp.int32),
            pltpu.VMEM((RCH,), jnp.int32),
            pltpu.VMEM((RCH,), jnp.int32),
            pltpu.VMEM((RCH, 128), jnp.float32),
            pltpu.VMEM((RCH, 128), jnp.float32),
            pltpu.SemaphoreType.DMA,
            pltpu.SemaphoreType.DMA,
            pltpu.SemaphoreType.DMA,
            pltpu.SemaphoreType.DMA,
            pltpu.SemaphoreType.DMA,
            pltpu.SemaphoreType.DMA,
        ],
    )


def _tcd_body(g_ref, w_ref, o_ref):
    o_ref[...] = jnp.dot(g_ref[...], w_ref[...],
                         preferred_element_type=jnp.float32)


_tcd = pl.pallas_call(
    _tcd_body,
    grid=(NBLK,),
    in_specs=[
        pl.BlockSpec((MB, 27 * 128), lambda b: (b, 0)),
        pl.BlockSpec((27 * 128, 64), lambda b: (0, 0)),
    ],
    out_specs=pl.BlockSpec((MB, 64), lambda b: (b, 0)),
    out_shape=jax.ShapeDtypeStruct((N, 64), jnp.float32),
    compiler_params=pltpu.CompilerParams(
        dimension_semantics=("arbitrary",)),
)


def kernel(features, depth, weight):
    d2 = depth[0]
    dpad = jnp.concatenate([d2[:, :1], d2], axis=1)
    feats_pad = jnp.concatenate(
        [jnp.concatenate([features, jnp.zeros((NF - N, C), jnp.float32)],
                         axis=0),
         jnp.zeros((NF, 128 - C), jnp.float32)], axis=1)
    wpad = jnp.concatenate(
        [weight, jnp.zeros((27, 128 - C, C), jnp.float32)],
        axis=1).reshape(27 * 128, C)

    v2d, nbv3 = _tca(dpad)
    vp = jnp.concatenate(
        [v2d.reshape(N), jnp.full((NP - N,), VPAD, jnp.int32)])
    padsent = SENTR + (jnp.arange(27 * (NP - N), dtype=jnp.int32) & 1023)
    nbvp = jnp.concatenate(
        [nbv3.reshape(27, N), padsent.reshape(27, NP - N)],
        axis=1).reshape(R)

    mesh = _mesh()
    grid = _scb(mesh)(vp)
    gmat = _scc(mesh)(nbvp, grid, feats_pad)
    return _tcd(gmat, wpad)


# speedup vs baseline: 1.0652x; 1.0652x over previous
import functools

import jax
import jax.numpy as jnp
from jax import lax
from jax.experimental import pallas as pl
from jax.experimental.pallas import tpu as pltpu
from jax.experimental.pallas import tpu_sc as plsc

H = 224
W = 224
NCOL = 225
N = H * NCOL
NP = 50688
C = 64
G = 65
G3 = G * G * G
GP = 276480
GCH = GP // 32
SENTR = 274688
VPAD = G3 + 7
NROWZ = N
NZROWS = 512
NF = N + NZROWS

OFFS = [(dx, dy, dz) for dx in (-1, 0, 1) for dy in (-1, 0, 1) for dz in (-1, 0, 1)]

R = 27 * NP
RCH = 512
CPK = NP // RCH
NCHUNK = R // RCH

MB = 512
NBLK = NP // MB


def _tca_body(d_ref, v_ref, nbv_ref):
    d = d_ref[...]
    zmin = jnp.min(d)
    zmax = jnp.max(d)
    z = (d - zmin) / (zmax - zmin + 1e-08)
    jj = lax.broadcasted_iota(jnp.int32, (H, NCOL), 1)
    ii = lax.broadcasted_iota(jnp.int32, (H, NCOL), 0)
    xf = (jj - 1).astype(jnp.float32) / (W - 1)
    yf = ii.astype(jnp.float32) / (H - 1)
    xb = jnp.clip(jnp.round(xf * 64).astype(jnp.int32), 0, 64)
    yb = jnp.clip(jnp.round(yf * 64).astype(jnp.int32), 0, 64)
    zb = jnp.clip(jnp.round(z * 64).astype(jnp.int32), 0, 64)
    iscls = jj == 0
    zero = jnp.zeros((H, NCOL), jnp.int32)
    xb = jnp.where(iscls, zero, xb)
    yb = jnp.where(iscls, zero, yb)
    zb = jnp.where(iscls, zero, zb)
    v = xb * 4225 + yb * 65 + zb
    v_ref[...] = v
    sent = SENTR + ((ii * NCOL + jj) & 1023)
    for k, (dx, dy, dz) in enumerate(OFFS):
        nx, ny, nz = xb + dx, yb + dy, zb + dz
        valid = ((nx >= 0) & (nx <= 64) & (ny >= 0) & (ny <= 64)
                 & (nz >= 0) & (nz <= 64))
        nbv_ref[k] = jnp.where(valid, v + (dx * 4225 + dy * 65 + dz), sent)


_tca = pl.pallas_call(
    _tca_body,
    out_shape=(
        jax.ShapeDtypeStruct((H, NCOL), jnp.int32),
        jax.ShapeDtypeStruct((27, H, NCOL), jnp.int32),
    ),
)


def _mesh():
    return plsc.VectorSubcoreMesh(core_axis_name="c", subcore_axis_name="s")


def _wid():
    return lax.axis_index("s") * 2 + lax.axis_index("c")


def _scb_body(v_hbm, grid_hbm, v_vmem, gch, sem):
    wid = _wid()
    lo = pl.multiple_of(wid * GCH, GCH)
    pltpu.sync_copy(v_hbm, v_vmem)
    neg1 = jnp.full((16,), -1, jnp.int32)

    def ms(i, c):
        gch[pl.ds(i * 16, 16)] = neg1
        return c

    lax.fori_loop(0, GCH // 16, ms, 0)

    lane = lax.iota(jnp.int32, 16)
    lane_bit = jnp.int32(1) << lane
    ones = jnp.full((16,), 1, jnp.int32)

    def body(i, c):
        vv = v_vmem[pl.ds(i * 16, 16)]
        m = (vv >= lo) & (vv < lo + GCH)
        loc = jnp.where(m, vv - lo, 0)
        base = ones * (i * 65536)
        plsc.store_scatter(gch, [loc], base, mask=m)
        plsc.addupdate_scatter(gch, [loc], lane_bit, mask=m)
        return c

    lax.fori_loop(0, NP // 16, body, 0)

    def decode(i, c):
        s = gch[pl.ds(i * 16, 16)]
        hi = lax.shift_right_logical(s, 16)
        lb = s & 0xFFFF
        f = lb.astype(jnp.float32)
        fbits = lax.bitcast_convert_type(f, jnp.int32)
        msb = lax.shift_right_logical(fbits, 23) - 127
        gch[pl.ds(i * 16, 16)] = jnp.where(s < 0, s, hi * 16 + msb)
        return c

    lax.fori_loop(0, GCH // 16, decode, 0)
    pltpu.sync_copy(gch, grid_hbm.at[pl.ds(lo, GCH)])


def _scb(mesh):
    return pl.kernel(
        _scb_body,
        out_type=jax.ShapeDtypeStruct((GP,), jnp.int32),
        mesh=mesh,
        compiler_params=pltpu.CompilerParams(needs_layout_passes=False),
        scratch_types=[
            pltpu.VMEM((NP,), jnp.int32),
            pltpu.VMEM((GCH,), jnp.int32),
            pltpu.SemaphoreType.DMA,
        ],
    )


def _coords(cc):
    k = cc // CPK
    n0 = pl.multiple_of((cc % CPK) * RCH, RCH)
    kc = pl.multiple_of(k * 128, 128)
    r0 = pl.multiple_of(cc * RCH, RCH)
    return n0, kc, r0


def _sel(rowv):
    lane = lax.iota(jnp.int32, 16)

    def body(i, c):
        pj = rowv[pl.ds(i * 16, 16)]
        zspread = NROWZ + i * 16 + lane
        rowv[pl.ds(i * 16, 16)] = jnp.where(pj < 0, zspread, pj)
        return c

    lax.fori_loop(0, RCH // 16, body, 0)


def _scc_body(nbv_hbm, grid_hbm, feats_hbm, g_hbm, idxv, rowv, gv, sem):
    wid = _wid()
    nc = 83 + (wid < NCHUNK - 83 * 32).astype(jnp.int32)

    def chunk(ci, c):
        cc = wid + 32 * ci
        n0, kc, r0 = _coords(cc)
        pltpu.sync_copy(nbv_hbm.at[pl.ds(r0, RCH)], idxv)
        pltpu.async_copy(grid_hbm.at[idxv], rowv, sem).wait()
        _sel(rowv)
        pltpu.async_copy(feats_hbm.at[rowv], gv, sem).wait()
        pltpu.sync_copy(gv, g_hbm.at[pl.ds(n0, RCH), pl.ds(kc, 128)])
        return c

    lax.fori_loop(0, nc, chunk, 0)


def _scc(mesh):
    return pl.kernel(
        _scc_body,
        out_type=jax.ShapeDtypeStruct((NP, 27 * 128), jnp.float32),
        mesh=mesh,
        compiler_params=pltpu.CompilerParams(needs_layout_passes=False),
        scratch_types=[
            pltpu.VMEM((RCH,), jnp.int32),
            pltpu.VMEM((RCH,), jnp.int32),
            pltpu.VMEM((RCH, 128), jnp.float32),
            pltpu.SemaphoreType.DMA,
        ],
    )


def _tcd_body(g_ref, w_ref, o_ref):
    o_ref[...] = jnp.dot(g_ref[...], w_ref[...],
                         preferred_element_type=jnp.float32)


_tcd = pl.pallas_call(
    _tcd_body,
    grid=(NBLK,),
    in_specs=[
        pl.BlockSpec((MB, 27 * 128), lambda b: (b, 0)),
        pl.BlockSpec((27 * 128, 64), lambda b: (0, 0)),
    ],
    out_specs=pl.BlockSpec((MB, 64), lambda b: (b, 0)),
    out_shape=jax.ShapeDtypeStruct((N, 64), jnp.float32),
    compiler_params=pltpu.CompilerParams(
        dimension_semantics=("arbitrary",)),
)


def kernel(features, depth, weight):
    d2 = depth[0]
    dpad = jnp.concatenate([d2[:, :1], d2], axis=1)
    feats_pad = jnp.concatenate(
        [jnp.concatenate([features, jnp.zeros((NF - N, C), jnp.float32)],
                         axis=0),
         jnp.zeros((NF, 128 - C), jnp.float32)], axis=1)
    wpad = jnp.concatenate(
        [weight, jnp.zeros((27, 128 - C, C), jnp.float32)],
        axis=1).reshape(27 * 128, C)

    v2d, nbv3 = _tca(dpad)
    vp = jnp.concatenate(
        [v2d.reshape(N), jnp.full((NP - N,), VPAD, jnp.int32)])
    padsent = SENTR + (jnp.arange(27 * (NP - N), dtype=jnp.int32) & 1023)
    nbvp = jnp.concatenate(
        [nbv3.reshape(27, N), padsent.reshape(27, NP - N)],
        axis=1).reshape(R)

    mesh = _mesh()
    grid = _scb(mesh)(vp)
    gmat = _scc(mesh)(nbvp, grid, feats_pad)
    return _tcd(gmat, wpad)


# speedup vs baseline: 1.1225x; 1.0539x over previous
import functools

import jax
import jax.numpy as jnp
from jax import lax
from jax.experimental import pallas as pl
from jax.experimental.pallas import tpu as pltpu
from jax.experimental.pallas import tpu_sc as plsc

H = 224
W = 224
NCOL = 225
N = H * NCOL
NP = 50688
C = 64
G = 65
G3 = G * G * G
GP = 276480
GCH = GP // 32
SENTR = 274688
VPAD = G3 + 7
NROWZ = N
NZROWS = 512
NF = N + NZROWS

OFFS = [(dx, dy, dz) for dx in (-1, 0, 1) for dy in (-1, 0, 1) for dz in (-1, 0, 1)]

R = 27 * NP
RCH = 384
CPK = NP // RCH
NCHUNK = R // RCH
NPAIR = (NCHUNK + 63) // 64

MB = 512
NBLK = NP // MB


def _tca_body(d_ref, v_ref, nbv_ref):
    d = d_ref[...]
    zmin = jnp.min(d)
    zmax = jnp.max(d)
    z = (d - zmin) / (zmax - zmin + 1e-08)
    jj = lax.broadcasted_iota(jnp.int32, (H, NCOL), 1)
    ii = lax.broadcasted_iota(jnp.int32, (H, NCOL), 0)
    xf = (jj - 1).astype(jnp.float32) / (W - 1)
    yf = ii.astype(jnp.float32) / (H - 1)
    xb = jnp.clip(jnp.round(xf * 64).astype(jnp.int32), 0, 64)
    yb = jnp.clip(jnp.round(yf * 64).astype(jnp.int32), 0, 64)
    zb = jnp.clip(jnp.round(z * 64).astype(jnp.int32), 0, 64)
    iscls = jj == 0
    zero = jnp.zeros((H, NCOL), jnp.int32)
    xb = jnp.where(iscls, zero, xb)
    yb = jnp.where(iscls, zero, yb)
    zb = jnp.where(iscls, zero, zb)
    v = xb * 4225 + yb * 65 + zb
    v_ref[...] = v
    sent = SENTR + ((ii * NCOL + jj) & 1023)
    for k, (dx, dy, dz) in enumerate(OFFS):
        nx, ny, nz = xb + dx, yb + dy, zb + dz
        valid = ((nx >= 0) & (nx <= 64) & (ny >= 0) & (ny <= 64)
                 & (nz >= 0) & (nz <= 64))
        nbv_ref[k] = jnp.where(valid, v + (dx * 4225 + dy * 65 + dz), sent)


_tca = pl.pallas_call(
    _tca_body,
    out_shape=(
        jax.ShapeDtypeStruct((H, NCOL), jnp.int32),
        jax.ShapeDtypeStruct((27, H, NCOL), jnp.int32),
    ),
)


def _mesh():
    return plsc.VectorSubcoreMesh(core_axis_name="c", subcore_axis_name="s")


def _wid():
    return lax.axis_index("s") * 2 + lax.axis_index("c")


def _scb_body(v_hbm, grid_hbm, v_vmem, gch, sem):
    wid = _wid()
    lo = pl.multiple_of(wid * GCH, GCH)
    pltpu.sync_copy(v_hbm, v_vmem)
    neg1 = jnp.full((16,), -1, jnp.int32)

    def ms(i, c):
        gch[pl.ds(i * 16, 16)] = neg1
        return c

    lax.fori_loop(0, GCH // 16, ms, 0)

    lane = lax.iota(jnp.int32, 16)
    lane_bit = jnp.int32(1) << lane
    ones = jnp.full((16,), 1, jnp.int32)

    def body(i, c):
        vv = v_vmem[pl.ds(i * 16, 16)]
        m = (vv >= lo) & (vv < lo + GCH)
        loc = jnp.where(m, vv - lo, 0)
        base = ones * (i * 65536)
        plsc.store_scatter(gch, [loc], base, mask=m)
        plsc.addupdate_scatter(gch, [loc], lane_bit, mask=m)
        return c

    lax.fori_loop(0, NP // 16, body, 0)

    def decode(i, c):
        s = gch[pl.ds(i * 16, 16)]
        hi = lax.shift_right_logical(s, 16)
        lb = s & 0xFFFF
        f = lb.astype(jnp.float32)
        fbits = lax.bitcast_convert_type(f, jnp.int32)
        msb = lax.shift_right_logical(fbits, 23) - 127
        gch[pl.ds(i * 16, 16)] = jnp.where(s < 0, s, hi * 16 + msb)
        return c

    lax.fori_loop(0, GCH // 16, decode, 0)
    pltpu.sync_copy(gch, grid_hbm.at[pl.ds(lo, GCH)])


def _scb(mesh):
    return pl.kernel(
        _scb_body,
        out_type=jax.ShapeDtypeStruct((GP,), jnp.int32),
        mesh=mesh,
        compiler_params=pltpu.CompilerParams(needs_layout_passes=False),
        scratch_types=[
            pltpu.VMEM((NP,), jnp.int32),
            pltpu.VMEM((GCH,), jnp.int32),
            pltpu.SemaphoreType.DMA,
        ],
    )


def _coords(cc):
    k = cc // CPK
    n0 = pl.multiple_of((cc % CPK) * RCH, RCH)
    kc = pl.multiple_of(k * 128, 128)
    r0 = pl.multiple_of(cc * RCH, RCH)
    return n0, kc, r0


def _sel(rowv):
    lane = lax.iota(jnp.int32, 16)

    def body(i, c):
        pj = rowv[pl.ds(i * 16, 16)]
        zspread = NROWZ + i * 16 + lane
        rowv[pl.ds(i * 16, 16)] = jnp.where(pj < 0, zspread, pj)
        return c

    lax.fori_loop(0, RCH // 16, body, 0)


def _scc_body(nbv_hbm, grid_hbm, feats_hbm, g_hbm,
              idxv, rowa, rowb, gva, gvb, sp, sfa, sfb, swa, swb):
    wid = _wid()

    def pair(i, c):
        c0 = jnp.minimum(wid + 32 * (2 * i), NCHUNK - 1)
        c1 = jnp.minimum(wid + 32 * (2 * i + 1), NCHUNK - 1)
        n00, kc0, r00 = _coords(c0)
        n01, kc1, r01 = _coords(c1)
        pltpu.sync_copy(nbv_hbm.at[pl.ds(r00, RCH)], idxv)
        pltpu.async_copy(grid_hbm.at[idxv], rowa, sp).wait()
        _sel(rowa)
        fa = pltpu.async_copy(feats_hbm.at[rowa], gva, sfa)
        pltpu.sync_copy(nbv_hbm.at[pl.ds(r01, RCH)], idxv)
        pltpu.async_copy(grid_hbm.at[idxv], rowb, sp).wait()
        _sel(rowb)
        fa.wait()
        wa = pltpu.async_copy(gva, g_hbm.at[pl.ds(n00, RCH), pl.ds(kc0, 128)],
                              swa)
        fb = pltpu.async_copy(feats_hbm.at[rowb], gvb, sfb)
        fb.wait()
        wb = pltpu.async_copy(gvb, g_hbm.at[pl.ds(n01, RCH), pl.ds(kc1, 128)],
                              swb)
        wa.wait()
        wb.wait()
        return c

    lax.fori_loop(0, NPAIR, pair, 0)


def _scc(mesh):
    return pl.kernel(
        _scc_body,
        out_type=jax.ShapeDtypeStruct((NP, 27 * 128), jnp.float32),
        mesh=mesh,
        compiler_params=pltpu.CompilerParams(needs_layout_passes=False),
        scratch_types=[
            pltpu.VMEM((RCH,), jnp.int32),
            pltpu.VMEM((RCH,), jnp.int32),
            pltpu.VMEM((RCH,), jnp.int32),
            pltpu.VMEM((RCH, 128), jnp.float32),
            pltpu.VMEM((RCH, 128), jnp.float32),
            pltpu.SemaphoreType.DMA,
            pltpu.SemaphoreType.DMA,
            pltpu.SemaphoreType.DMA,
            pltpu.SemaphoreType.DMA,
            pltpu.SemaphoreType.DMA,
        ],
    )


def _tcd_body(g_ref, w_ref, o_ref):
    o_ref[...] = jnp.dot(g_ref[...], w_ref[...],
                         preferred_element_type=jnp.float32)


_tcd = pl.pallas_call(
    _tcd_body,
    grid=(NBLK,),
    in_specs=[
        pl.BlockSpec((MB, 27 * 128), lambda b: (b, 0)),
        pl.BlockSpec((27 * 128, 64), lambda b: (0, 0)),
    ],
    out_specs=pl.BlockSpec((MB, 64), lambda b: (b, 0)),
    out_shape=jax.ShapeDtypeStruct((N, 64), jnp.float32),
    compiler_params=pltpu.CompilerParams(
        dimension_semantics=("arbitrary",)),
)


def kernel(features, depth, weight):
    d2 = depth[0]
    dpad = jnp.concatenate([d2[:, :1], d2], axis=1)
    feats_pad = jnp.concatenate(
        [jnp.concatenate([features, jnp.zeros((NF - N, C), jnp.float32)],
                         axis=0),
         jnp.zeros((NF, 128 - C), jnp.float32)], axis=1)
    wpad = jnp.concatenate(
        [weight, jnp.zeros((27, 128 - C, C), jnp.float32)],
        axis=1).reshape(27 * 128, C)

    v2d, nbv3 = _tca(dpad)
    vp = jnp.concatenate(
        [v2d.reshape(N), jnp.full((NP - N,), VPAD, jnp.int32)])
    padsent = SENTR + (jnp.arange(27 * (NP - N), dtype=jnp.int32) & 1023)
    nbvp = jnp.concatenate(
        [nbv3.reshape(27, N), padsent.reshape(27, NP - N)],
        axis=1).reshape(R)

    mesh = _mesh()
    grid = _scb(mesh)(vp)
    gmat = _scc(mesh)(nbvp, grid, feats_pad)
    return _tcd(gmat, wpad)


# speedup vs baseline: 1.1233x; 1.0006x over previous
import jax
import jax.numpy as jnp
from jax import lax
from jax.experimental import pallas as pl
from jax.experimental.pallas import tpu as pltpu
from jax.experimental.pallas import tpu_sc as plsc

H = 224
W = 224
NCOL = 225
N = H * NCOL
NP = 50688
C = 64
G = 65
G3 = G * G * G
GP = 276480
GCH = GP // 32
SENTR = 274688
VPAD = G3 + 7
NROWZ = N
NZROWS = 512
NF = N + NZROWS

OFFS = [(dx, dy, dz) for dx in (-1, 0, 1) for dy in (-1, 0, 1) for dz in (-1, 0, 1)]

R = 27 * NP
RCH = 384
CPK = NP // RCH
NCHUNK = R // RCH
NPAIR = (NCHUNK + 63) // 64

MB = 512
NBLK = NP // MB


def _tca_body(d_ref, v_ref, nbv_ref):
    d = d_ref[...]
    zmin = jnp.min(d)
    zmax = jnp.max(d)
    z = (d - zmin) / (zmax - zmin + 1e-08)
    jj = lax.broadcasted_iota(jnp.int32, (H, NCOL), 1)
    ii = lax.broadcasted_iota(jnp.int32, (H, NCOL), 0)
    xf = (jj - 1).astype(jnp.float32) / (W - 1)
    yf = ii.astype(jnp.float32) / (H - 1)
    xb = jnp.clip(jnp.round(xf * 64).astype(jnp.int32), 0, 64)
    yb = jnp.clip(jnp.round(yf * 64).astype(jnp.int32), 0, 64)
    zb = jnp.clip(jnp.round(z * 64).astype(jnp.int32), 0, 64)
    iscls = jj == 0
    zero = jnp.zeros((H, NCOL), jnp.int32)
    xb = jnp.where(iscls, zero, xb)
    yb = jnp.where(iscls, zero, yb)
    zb = jnp.where(iscls, zero, zb)
    v = xb * 4225 + yb * 65 + zb
    v_ref[...] = v
    sent = SENTR + ((ii * NCOL + jj) & 1023)
    for k, (dx, dy, dz) in enumerate(OFFS):
        nx, ny, nz = xb + dx, yb + dy, zb + dz
        valid = ((nx >= 0) & (nx <= 64) & (ny >= 0) & (ny <= 64)
                 & (nz >= 0) & (nz <= 64))
        nbv_ref[k] = jnp.where(valid, v + (dx * 4225 + dy * 65 + dz), sent)


_tca = pl.pallas_call(
    _tca_body,
    out_shape=(
        jax.ShapeDtypeStruct((H, NCOL), jnp.int32),
        jax.ShapeDtypeStruct((27, H, NCOL), jnp.int32),
    ),
)


def _mesh():
    return plsc.VectorSubcoreMesh(core_axis_name="c", subcore_axis_name="s")


def _wid():
    return lax.axis_index("s") * 2 + lax.axis_index("c")


def _scb_body(v_hbm, grid_hbm, v_vmem, gch, sem):
    wid = _wid()
    lo = pl.multiple_of(wid * GCH, GCH)
    pltpu.sync_copy(v_hbm, v_vmem)
    neg1 = jnp.full((16,), -1, jnp.int32)

    def ms(i, c):
        gch[pl.ds(i * 16, 16)] = neg1
        return c

    lax.fori_loop(0, GCH // 16, ms, 0)

    lane = lax.iota(jnp.int32, 16)
    lane_bit = jnp.int32(1) << lane
    ones = jnp.full((16,), 1, jnp.int32)

    def body(i, c):
        vv = v_vmem[pl.ds(i * 16, 16)]
        m = (vv >= lo) & (vv < lo + GCH)
        loc = jnp.where(m, vv - lo, 0)
        base = ones * (i * 65536)
        plsc.store_scatter(gch, [loc], base, mask=m)
        plsc.addupdate_scatter(gch, [loc], lane_bit, mask=m)
        return c

    lax.fori_loop(0, NP // 16, body, 0)

    def decode(i, c):
        s = gch[pl.ds(i * 16, 16)]
        hi = lax.shift_right_logical(s, 16)
        lb = s & 0xFFFF
        f = lb.astype(jnp.float32)
        fbits = lax.bitcast_convert_type(f, jnp.int32)
        msb = lax.shift_right_logical(fbits, 23) - 127
        gch[pl.ds(i * 16, 16)] = jnp.where(s < 0, s, hi * 16 + msb)
        return c

    lax.fori_loop(0, GCH // 16, decode, 0)
    pltpu.sync_copy(gch, grid_hbm.at[pl.ds(lo, GCH)])


def _scb(mesh):
    return pl.kernel(
        _scb_body,
        out_type=jax.ShapeDtypeStruct((GP,), jnp.int32),
        mesh=mesh,
        compiler_params=pltpu.CompilerParams(needs_layout_passes=False),
        scratch_types=[
            pltpu.VMEM((NP,), jnp.int32),
            pltpu.VMEM((GCH,), jnp.int32),
            pltpu.SemaphoreType.DMA,
        ],
    )


def _coords(cc):
    k = cc // CPK
    n0 = pl.multiple_of((cc % CPK) * RCH, RCH)
    kc = pl.multiple_of(k * 128, 128)
    r0 = pl.multiple_of(cc * RCH, RCH)
    return n0, kc, r0


def _sel(rowv):
    lane = lax.iota(jnp.int32, 16)

    def body(i, c):
        pj = rowv[pl.ds(i * 16, 16)]
        zspread = NROWZ + i * 16 + lane
        rowv[pl.ds(i * 16, 16)] = jnp.where(pj < 0, zspread, pj)
        return c

    lax.fori_loop(0, RCH // 16, body, 0)


def _scc_body(nbv_hbm, grid_hbm, feats_hbm, g_hbm,
              idxv, rowa, rowb, gva, gvb, sp, sfa, sfb, swa, swb):
    wid = _wid()

    def pair(i, c):
        c0 = jnp.minimum(wid + 32 * (2 * i), NCHUNK - 1)
        c1 = jnp.minimum(wid + 32 * (2 * i + 1), NCHUNK - 1)
        n00, kc0, r00 = _coords(c0)
        n01, kc1, r01 = _coords(c1)
        pltpu.sync_copy(nbv_hbm.at[pl.ds(r00, RCH)], idxv)
        pltpu.async_copy(grid_hbm.at[idxv], rowa, sp).wait()
        _sel(rowa)
        fa = pltpu.async_copy(feats_hbm.at[rowa], gva, sfa)
        pltpu.sync_copy(nbv_hbm.at[pl.ds(r01, RCH)], idxv)
        pltpu.async_copy(grid_hbm.at[idxv], rowb, sp).wait()
        _sel(rowb)
        fa.wait()
        wa = pltpu.async_copy(gva, g_hbm.at[pl.ds(n00, RCH), pl.ds(kc0, 128)],
                              swa)
        fb = pltpu.async_copy(feats_hbm.at[rowb], gvb, sfb)
        fb.wait()
        wb = pltpu.async_copy(gvb, g_hbm.at[pl.ds(n01, RCH), pl.ds(kc1, 128)],
                              swb)
        wa.wait()
        wb.wait()
        return c

    lax.fori_loop(0, NPAIR, pair, 0)


def _scc(mesh):
    return pl.kernel(
        _scc_body,
        out_type=jax.ShapeDtypeStruct((NP, 27 * 128), jnp.float32),
        mesh=mesh,
        compiler_params=pltpu.CompilerParams(needs_layout_passes=False),
        scratch_types=[
            pltpu.VMEM((RCH,), jnp.int32),
            pltpu.VMEM((RCH,), jnp.int32),
            pltpu.VMEM((RCH,), jnp.int32),
            pltpu.VMEM((RCH, 128), jnp.float32),
            pltpu.VMEM((RCH, 128), jnp.float32),
            pltpu.SemaphoreType.DMA,
            pltpu.SemaphoreType.DMA,
            pltpu.SemaphoreType.DMA,
            pltpu.SemaphoreType.DMA,
            pltpu.SemaphoreType.DMA,
        ],
    )


def _tcd_body(g_ref, w_ref, o_ref):
    o_ref[...] = jnp.dot(g_ref[...], w_ref[...],
                         preferred_element_type=jnp.float32)


_tcd = pl.pallas_call(
    _tcd_body,
    grid=(NBLK,),
    in_specs=[
        pl.BlockSpec((MB, 27 * 128), lambda b: (b, 0)),
        pl.BlockSpec((27 * 128, 64), lambda b: (0, 0)),
    ],
    out_specs=pl.BlockSpec((MB, 64), lambda b: (b, 0)),
    out_shape=jax.ShapeDtypeStruct((N, 64), jnp.float32),
    compiler_params=pltpu.CompilerParams(
        dimension_semantics=("arbitrary",)),
)


def kernel(features, depth, weight):
    d2 = depth[0]
    dpad = jnp.concatenate([d2[:, :1], d2], axis=1)
    feats_pad = jnp.concatenate(
        [jnp.concatenate([features, jnp.zeros((NF - N, C), jnp.float32)],
                         axis=0),
         jnp.zeros((NF, 128 - C), jnp.float32)], axis=1)
    wpad = jnp.concatenate(
        [weight, jnp.zeros((27, 128 - C, C), jnp.float32)],
        axis=1).reshape(27 * 128, C)

    v2d, nbv3 = _tca(dpad)
    vp = jnp.concatenate(
        [v2d.reshape(N), jnp.full((NP - N,), VPAD, jnp.int32)])
    padsent = SENTR + (jnp.arange(27 * (NP - N), dtype=jnp.int32) & 1023)
    nbvp = jnp.concatenate(
        [nbv3.reshape(27, N), padsent.reshape(27, NP - N)],
        axis=1).reshape(R)

    mesh = _mesh()
    grid = _scb(mesh)(vp)
    gmat = _scc(mesh)(nbvp, grid, feats_pad)
    return _tcd(gmat, wpad)
